# baseline (device time: 328636 ns/iter reference)
import jax
import jax.numpy as jnp
from jax import lax
from jax.experimental import pallas as pl
from jax.experimental.pallas import tpu as pltpu

W = 32


def kernel(A, B):
    m, k = A.shape
    _, n = B.shape
    ch = m // W

    def body(a_ref, b_ref, out_ref, p_ref, recv_ref,
             rs_send_sems, rs_recv_sems, ag_send_sems, ag_recv_sems):
        r = lax.axis_index("i")
        left = lax.rem(r - 1 + W, W)
        right = lax.rem(r + 1, W)

        barrier_sem = pltpu.get_barrier_semaphore()
        for nbr in (left, right):
            pl.semaphore_signal(
                barrier_sem, inc=1,
                device_id=(nbr,), device_id_type=pl.DeviceIdType.MESH,
            )
        pl.semaphore_wait(barrier_sem, 2)

        a = a_ref[...].astype(jnp.bfloat16)
        b = b_ref[...].astype(jnp.bfloat16)
        p_ref[...] = jnp.dot(a, b, preferred_element_type=jnp.float32)

        for h in range(W - 1):
            s = lax.rem(r - h - 1 + W, W)
            if h > 0:
                p_ref[pl.ds(s * ch, ch), :] = (
                    p_ref[pl.ds(s * ch, ch), :] + recv_ref[h - 1]
                )
            rdma = pltpu.make_async_remote_copy(
                src_ref=p_ref.at[pl.ds(s * ch, ch), :],
                dst_ref=recv_ref.at[h],
                send_sem=rs_send_sems.at[h],
                recv_sem=rs_recv_sems.at[h],
                device_id=(right,),
                device_id_type=pl.DeviceIdType.MESH,
            )
            rdma.start()
            rdma.wait()

        z = p_ref[pl.ds(r * ch, ch), :] + recv_ref[W - 2]
        out_ref[pl.ds(r * ch, ch), :] = (
            0.5 * z * (1.0 + jnp.tanh(0.7978845608 * (z + 0.044715 * z * z * z)))
        )

        for g in range(W - 1):
            c = lax.rem(r - g + W, W)
            rdma = pltpu.make_async_remote_copy(
                src_ref=out_ref.at[pl.ds(c * ch, ch), :],
                dst_ref=out_ref.at[pl.ds(c * ch, ch), :],
                send_sem=ag_send_sems.at[g],
                recv_sem=ag_recv_sems.at[g],
                device_id=(right,),
                device_id_type=pl.DeviceIdType.MESH,
            )
            rdma.start()
            rdma.wait()

    return pl.pallas_call(
        body,
        out_shape=jax.ShapeDtypeStruct((m, n), jnp.float32),
        in_specs=[
            pl.BlockSpec(memory_space=pltpu.VMEM),
            pl.BlockSpec(memory_space=pltpu.VMEM),
        ],
        out_specs=pl.BlockSpec(memory_space=pltpu.VMEM),
        scratch_shapes=[
            pltpu.VMEM((m, n), jnp.float32),
            pltpu.VMEM((W - 1, ch, n), jnp.float32),
            pltpu.SemaphoreType.DMA((W - 1,)),
            pltpu.SemaphoreType.DMA((W - 1,)),
            pltpu.SemaphoreType.DMA((W - 1,)),
            pltpu.SemaphoreType.DMA((W - 1,)),
        ],
        compiler_params=pltpu.CompilerParams(collective_id=0),
    )(A, B)


# device time: 229454 ns/iter; 1.4323x vs baseline; 1.4323x over previous
import jax
import jax.numpy as jnp
from jax import lax
from jax.experimental import pallas as pl
from jax.experimental.pallas import tpu as pltpu

W = 32


def _gelu(z):
    return 0.5 * z * (1.0 + jnp.tanh(0.7978845608 * (z + 0.044715 * z * z * z)))


def kernel(A, B):
    m, k = A.shape
    _, n = B.shape
    ch = m // W

    def body(a_ref, b_ref, out_ref, p_ref, recv_ref, stage_ref,
             rs_send_sems, rs_recv_sems, ag_send_sems, ag_recv_sems):
        r = lax.axis_index("i")
        left = lax.rem(r - 1 + W, W)
        right = lax.rem(r + 1, W)

        barrier_sem = pltpu.get_barrier_semaphore()
        for nbr in (left, right):
            pl.semaphore_signal(
                barrier_sem, inc=1,
                device_id=(nbr,), device_id_type=pl.DeviceIdType.MESH,
            )
        pl.semaphore_wait(barrier_sem, 2)

        a = a_ref[...].astype(jnp.bfloat16)
        b = b_ref[...].astype(jnp.bfloat16)
        p_ref[...] = jnp.dot(a, b, preferred_element_type=jnp.float32)

        for h in range(W - 1):
            s = lax.rem(r - h - 1 + W, W)
            if h > 0:
                p_ref[pl.ds(s * ch, ch), :] = (
                    p_ref[pl.ds(s * ch, ch), :]
                    + recv_ref[h - 1].astype(jnp.float32)
                )
            stage_ref[...] = p_ref[pl.ds(s * ch, ch), :].astype(jnp.bfloat16)
            rdma = pltpu.make_async_remote_copy(
                src_ref=stage_ref,
                dst_ref=recv_ref.at[h],
                send_sem=rs_send_sems.at[h],
                recv_sem=rs_recv_sems.at[h],
                device_id=(right,),
                device_id_type=pl.DeviceIdType.MESH,
            )
            rdma.start()
            rdma.wait()

        z = p_ref[pl.ds(r * ch, ch), :] + recv_ref[W - 2].astype(jnp.float32)
        out_ref[pl.ds(r * ch, ch), :] = _gelu(z).astype(jnp.bfloat16)

        for g in range(W - 1):
            c = lax.rem(r - g + W, W)
            rdma = pltpu.make_async_remote_copy(
                src_ref=out_ref.at[pl.ds(c * ch, ch), :],
                dst_ref=out_ref.at[pl.ds(c * ch, ch), :],
                send_sem=ag_send_sems.at[g],
                recv_sem=ag_recv_sems.at[g],
                device_id=(right,),
                device_id_type=pl.DeviceIdType.MESH,
            )
            rdma.start()
            rdma.wait()

    return pl.pallas_call(
        body,
        out_shape=jax.ShapeDtypeStruct((m, n), jnp.bfloat16),
        in_specs=[
            pl.BlockSpec(memory_space=pltpu.VMEM),
            pl.BlockSpec(memory_space=pltpu.VMEM),
        ],
        out_specs=pl.BlockSpec(memory_space=pltpu.VMEM),
        scratch_shapes=[
            pltpu.VMEM((m, n), jnp.float32),
            pltpu.VMEM((W - 1, ch, n), jnp.bfloat16),
            pltpu.VMEM((ch, n), jnp.bfloat16),
            pltpu.SemaphoreType.DMA((W - 1,)),
            pltpu.SemaphoreType.DMA((W - 1,)),
            pltpu.SemaphoreType.DMA((W - 1,)),
            pltpu.SemaphoreType.DMA((W - 1,)),
        ],
        compiler_params=pltpu.CompilerParams(collective_id=0),
    )(A, B)


# device time: 152425 ns/iter; 2.1561x vs baseline; 1.5054x over previous
import jax
import jax.numpy as jnp
from jax import lax
from jax.experimental import pallas as pl
from jax.experimental.pallas import tpu as pltpu

W = 32
G = 4
S = 8


def _gelu(z):
    return 0.5 * z * (1.0 + jnp.tanh(0.7978845608 * (z + 0.044715 * z * z * z)))


def kernel(A, B):
    m, k = A.shape
    _, n = B.shape
    slab = m // S
    sub = slab // G

    def body(a_ref, b_ref, out_ref, p_ref, recv1_ref, recv2_ref,
             stage1_ref, stage2_ref,
             rs1_send, rs1_recv, rs2_send, rs2_recv,
             ag2_send, ag2_recv, ag1_send, ag1_recv):
        r = lax.axis_index("i")
        g = r // S
        j = lax.rem(r, S)
        right1 = g * S + lax.rem(j + 1, S)
        left1 = g * S + lax.rem(j + S - 1, S)
        right2 = lax.rem(g + 1, G) * S + j
        left2 = lax.rem(g + G - 1, G) * S + j

        barrier_sem = pltpu.get_barrier_semaphore()
        for nbr in (left1, right1, left2, right2):
            pl.semaphore_signal(
                barrier_sem, inc=1,
                device_id=(nbr,), device_id_type=pl.DeviceIdType.MESH,
            )
        pl.semaphore_wait(barrier_sem, 4)

        a = a_ref[...].astype(jnp.bfloat16)
        b = b_ref[...].astype(jnp.bfloat16)
        p_ref[...] = jnp.dot(a, b, preferred_element_type=jnp.float32)

        for h in range(S - 1):
            s1 = lax.rem(j + S - h - 1, S)
            if h > 0:
                p_ref[pl.ds(s1 * slab, slab), :] = (
                    p_ref[pl.ds(s1 * slab, slab), :]
                    + recv1_ref[h - 1].astype(jnp.float32)
                )
            stage1_ref[...] = p_ref[pl.ds(s1 * slab, slab), :].astype(jnp.bfloat16)
            rdma = pltpu.make_async_remote_copy(
                src_ref=stage1_ref,
                dst_ref=recv1_ref.at[h],
                send_sem=rs1_send.at[h],
                recv_sem=rs1_recv.at[h],
                device_id=(right1,),
                device_id_type=pl.DeviceIdType.MESH,
            )
            rdma.start()
            rdma.wait()
        p_ref[pl.ds(j * slab, slab), :] = (
            p_ref[pl.ds(j * slab, slab), :]
            + recv1_ref[S - 2].astype(jnp.float32)
        )

        base = j * slab
        for h in range(G - 1):
            s2 = lax.rem(g + G - h - 1, G)
            if h > 0:
                p_ref[pl.ds(base + s2 * sub, sub), :] = (
                    p_ref[pl.ds(base + s2 * sub, sub), :]
                    + recv2_ref[h - 1].astype(jnp.float32)
                )
            stage2_ref[...] = p_ref[pl.ds(base + s2 * sub, sub), :].astype(jnp.bfloat16)
            rdma = pltpu.make_async_remote_copy(
                src_ref=stage2_ref,
                dst_ref=recv2_ref.at[h],
                send_sem=rs2_send.at[h],
                recv_sem=rs2_recv.at[h],
                device_id=(right2,),
                device_id_type=pl.DeviceIdType.MESH,
            )
            rdma.start()
            rdma.wait()

        z = (
            p_ref[pl.ds(base + g * sub, sub), :]
            + recv2_ref[G - 2].astype(jnp.float32)
        )
        out_ref[pl.ds(base + g * sub, sub), :] = _gelu(z).astype(jnp.bfloat16)

        for hh in range(G - 1):
            c2 = lax.rem(g + G - hh, G)
            rdma = pltpu.make_async_remote_copy(
                src_ref=out_ref.at[pl.ds(base + c2 * sub, sub), :],
                dst_ref=out_ref.at[pl.ds(base + c2 * sub, sub), :],
                send_sem=ag2_send.at[hh],
                recv_sem=ag2_recv.at[hh],
                device_id=(right2,),
                device_id_type=pl.DeviceIdType.MESH,
            )
            rdma.start()
            rdma.wait()

        for hh in range(S - 1):
            c1 = lax.rem(j + S - hh, S)
            rdma = pltpu.make_async_remote_copy(
                src_ref=out_ref.at[pl.ds(c1 * slab, slab), :],
                dst_ref=out_ref.at[pl.ds(c1 * slab, slab), :],
                send_sem=ag1_send.at[hh],
                recv_sem=ag1_recv.at[hh],
                device_id=(right1,),
                device_id_type=pl.DeviceIdType.MESH,
            )
            rdma.start()
            rdma.wait()

    return pl.pallas_call(
        body,
        out_shape=jax.ShapeDtypeStruct((m, n), jnp.bfloat16),
        in_specs=[
            pl.BlockSpec(memory_space=pltpu.VMEM),
            pl.BlockSpec(memory_space=pltpu.VMEM),
        ],
        out_specs=pl.BlockSpec(memory_space=pltpu.VMEM),
        scratch_shapes=[
            pltpu.VMEM((m, n), jnp.float32),
            pltpu.VMEM((S - 1, slab, n), jnp.bfloat16),
            pltpu.VMEM((G - 1, sub, n), jnp.bfloat16),
            pltpu.VMEM((slab, n), jnp.bfloat16),
            pltpu.VMEM((sub, n), jnp.bfloat16),
            pltpu.SemaphoreType.DMA((S - 1,)),
            pltpu.SemaphoreType.DMA((S - 1,)),
            pltpu.SemaphoreType.DMA((G - 1,)),
            pltpu.SemaphoreType.DMA((G - 1,)),
            pltpu.SemaphoreType.DMA((G - 1,)),
            pltpu.SemaphoreType.DMA((G - 1,)),
            pltpu.SemaphoreType.DMA((S - 1,)),
            pltpu.SemaphoreType.DMA((S - 1,)),
        ],
        compiler_params=pltpu.CompilerParams(collective_id=0),
    )(A, B)


# device time: 111192 ns/iter; 2.9556x vs baseline; 1.3708x over previous
import jax
import jax.numpy as jnp
from jax import lax
from jax.experimental import pallas as pl
from jax.experimental.pallas import tpu as pltpu

W = 32
G = 4
S = 8

PERM = (0, 1, 2, 5, 6, 7, 4, 3)
INV = (0, 1, 2, 7, 6, 3, 4, 5)


def _gelu(z):
    return 0.5 * z * (1.0 + jnp.tanh(0.7978845608 * (z + 0.044715 * z * z * z)))


def kernel(A, B):
    m, k = A.shape
    _, n = B.shape
    slab = m // S
    half = slab // 2
    sub = slab // G

    r = lax.axis_index("i")
    g = r // S
    j = lax.rem(r, S)
    perm = jnp.array(PERM, dtype=jnp.int32)
    inv = jnp.array(INV, dtype=jnp.int32)
    q = inv[j]
    right1 = g * S + perm[lax.rem(q + 1, S)]
    left1 = g * S + perm[lax.rem(q + S - 1, S)]
    right2 = lax.rem(g + 1, G) * S + j
    left2 = lax.rem(g + G - 1, G) * S + j
    scal = jnp.stack([q, right1, left1, right2, left2]).astype(jnp.int32)

    def body(scal_ref, a_ref, b_ref, out_ref, p_ref,
             recv1r_ref, recv1l_ref, recv2_ref,
             stage1r_ref, stage1l_ref, stage2_ref,
             rs1r_send, rs1r_recv, rs1l_send, rs1l_recv,
             rs2_send, rs2_recv, ag2_send, ag2_recv,
             ag1r_send, ag1r_recv, ag1l_send, ag1l_recv):
        gg = lax.axis_index("i") // S
        q = scal_ref[0]
        right1 = scal_ref[1]
        left1 = scal_ref[2]
        right2 = scal_ref[3]
        left2 = scal_ref[4]

        barrier_sem = pltpu.get_barrier_semaphore()
        for nbr in (left1, right1, left2, right2):
            pl.semaphore_signal(
                barrier_sem, inc=1,
                device_id=(nbr,), device_id_type=pl.DeviceIdType.MESH,
            )
        pl.semaphore_wait(barrier_sem, 4)

        a = a_ref[...].astype(jnp.bfloat16)
        b = b_ref[...].astype(jnp.bfloat16)
        p_ref[...] = jnp.dot(a, b, preferred_element_type=jnp.float32)

        for h in range(S - 1):
            sr = lax.rem(q + S - h - 1, S)
            sl = lax.rem(q + h + 1, S)
            if h > 0:
                p_ref[pl.ds(sr * slab, half), :] = (
                    p_ref[pl.ds(sr * slab, half), :]
                    + recv1r_ref[h - 1].astype(jnp.float32)
                )
                p_ref[pl.ds(sl * slab + half, half), :] = (
                    p_ref[pl.ds(sl * slab + half, half), :]
                    + recv1l_ref[h - 1].astype(jnp.float32)
                )
            stage1r_ref[...] = p_ref[pl.ds(sr * slab, half), :].astype(jnp.bfloat16)
            stage1l_ref[...] = p_ref[pl.ds(sl * slab + half, half), :].astype(jnp.bfloat16)
            rdma_r = pltpu.make_async_remote_copy(
                src_ref=stage1r_ref,
                dst_ref=recv1r_ref.at[h],
                send_sem=rs1r_send.at[h],
                recv_sem=rs1r_recv.at[h],
                device_id=(right1,),
                device_id_type=pl.DeviceIdType.MESH,
            )
            rdma_l = pltpu.make_async_remote_copy(
                src_ref=stage1l_ref,
                dst_ref=recv1l_ref.at[h],
                send_sem=rs1l_send.at[h],
                recv_sem=rs1l_recv.at[h],
                device_id=(left1,),
                device_id_type=pl.DeviceIdType.MESH,
            )
            rdma_r.start()
            rdma_l.start()
            rdma_r.wait()
            rdma_l.wait()
        p_ref[pl.ds(q * slab, half), :] = (
            p_ref[pl.ds(q * slab, half), :]
            + recv1r_ref[S - 2].astype(jnp.float32)
        )
        p_ref[pl.ds(q * slab + half, half), :] = (
            p_ref[pl.ds(q * slab + half, half), :]
            + recv1l_ref[S - 2].astype(jnp.float32)
        )

        base = q * slab
        for h in range(G - 1):
            s2 = lax.rem(gg + G - h - 1, G)
            if h > 0:
                p_ref[pl.ds(base + s2 * sub, sub), :] = (
                    p_ref[pl.ds(base + s2 * sub, sub), :]
                    + recv2_ref[h - 1].astype(jnp.float32)
                )
            stage2_ref[...] = p_ref[pl.ds(base + s2 * sub, sub), :].astype(jnp.bfloat16)
            rdma = pltpu.make_async_remote_copy(
                src_ref=stage2_ref,
                dst_ref=recv2_ref.at[h],
                send_sem=rs2_send.at[h],
                recv_sem=rs2_recv.at[h],
                device_id=(right2,),
                device_id_type=pl.DeviceIdType.MESH,
            )
            rdma.start()
            rdma.wait()

        z = (
            p_ref[pl.ds(base + gg * sub, sub), :]
            + recv2_ref[G - 2].astype(jnp.float32)
        )
        out_ref[pl.ds(base + gg * sub, sub), :] = _gelu(z).astype(jnp.bfloat16)

        for hh in range(G - 1):
            c2 = lax.rem(gg + G - hh, G)
            rdma = pltpu.make_async_remote_copy(
                src_ref=out_ref.at[pl.ds(base + c2 * sub, sub), :],
                dst_ref=out_ref.at[pl.ds(base + c2 * sub, sub), :],
                send_sem=ag2_send.at[hh],
                recv_sem=ag2_recv.at[hh],
                device_id=(right2,),
                device_id_type=pl.DeviceIdType.MESH,
            )
            rdma.start()
            rdma.wait()

        for hh in range(S - 1):
            cr = lax.rem(q + S - hh, S)
            cl = lax.rem(q + hh, S)
            rdma_r = pltpu.make_async_remote_copy(
                src_ref=out_ref.at[pl.ds(cr * slab, half), :],
                dst_ref=out_ref.at[pl.ds(cr * slab, half), :],
                send_sem=ag1r_send.at[hh],
                recv_sem=ag1r_recv.at[hh],
                device_id=(right1,),
                device_id_type=pl.DeviceIdType.MESH,
            )
            rdma_l = pltpu.make_async_remote_copy(
                src_ref=out_ref.at[pl.ds(cl * slab + half, half), :],
                dst_ref=out_ref.at[pl.ds(cl * slab + half, half), :],
                send_sem=ag1l_send.at[hh],
                recv_sem=ag1l_recv.at[hh],
                device_id=(left1,),
                device_id_type=pl.DeviceIdType.MESH,
            )
            rdma_r.start()
            rdma_l.start()
            rdma_r.wait()
            rdma_l.wait()

    return pl.pallas_call(
        body,
        out_shape=jax.ShapeDtypeStruct((m, n), jnp.bfloat16),
        in_specs=[
            pl.BlockSpec(memory_space=pltpu.SMEM),
            pl.BlockSpec(memory_space=pltpu.VMEM),
            pl.BlockSpec(memory_space=pltpu.VMEM),
        ],
        out_specs=pl.BlockSpec(memory_space=pltpu.VMEM),
        scratch_shapes=[
            pltpu.VMEM((m, n), jnp.float32),
            pltpu.VMEM((S - 1, half, n), jnp.bfloat16),
            pltpu.VMEM((S - 1, half, n), jnp.bfloat16),
            pltpu.VMEM((G - 1, sub, n), jnp.bfloat16),
            pltpu.VMEM((half, n), jnp.bfloat16),
            pltpu.VMEM((half, n), jnp.bfloat16),
            pltpu.VMEM((sub, n), jnp.bfloat16),
            pltpu.SemaphoreType.DMA((S - 1,)),
            pltpu.SemaphoreType.DMA((S - 1,)),
            pltpu.SemaphoreType.DMA((S - 1,)),
            pltpu.SemaphoreType.DMA((S - 1,)),
            pltpu.SemaphoreType.DMA((G - 1,)),
            pltpu.SemaphoreType.DMA((G - 1,)),
            pltpu.SemaphoreType.DMA((G - 1,)),
            pltpu.SemaphoreType.DMA((G - 1,)),
            pltpu.SemaphoreType.DMA((S - 1,)),
            pltpu.SemaphoreType.DMA((S - 1,)),
            pltpu.SemaphoreType.DMA((S - 1,)),
            pltpu.SemaphoreType.DMA((S - 1,)),
        ],
        compiler_params=pltpu.CompilerParams(collective_id=0),
    )(scal, A, B)


# device time: 90334 ns/iter; 3.6380x vs baseline; 1.2309x over previous
import jax
import jax.numpy as jnp
from jax import lax
from jax.experimental import pallas as pl
from jax.experimental.pallas import tpu as pltpu

W = 32
G = 4
S = 8
Q = 3

PERM = (0, 1, 2, 5, 6, 7, 4, 3)
INV = (0, 1, 2, 7, 6, 3, 4, 5)


def _gelu(z):
    return 0.5 * z * (1.0 + jnp.tanh(0.7978845608 * (z + 0.044715 * z * z * z)))


def kernel(A, B):
    m, k = A.shape
    _, n = B.shape
    slab = m // S
    half = slab // 2
    qr = half // Q
    sub = slab // G

    r = lax.axis_index("i")
    g = r // S
    j = lax.rem(r, S)
    perm = jnp.array(PERM, dtype=jnp.int32)
    inv = jnp.array(INV, dtype=jnp.int32)
    q = inv[j]
    right1 = g * S + perm[lax.rem(q + 1, S)]
    left1 = g * S + perm[lax.rem(q + S - 1, S)]
    right2 = lax.rem(g + 1, G) * S + j
    left2 = lax.rem(g + G - 1, G) * S + j
    scal = jnp.stack([q, right1, left1, right2, left2]).astype(jnp.int32)

    def body(scal_ref, a_ref, b_ref, out_ref, p_ref, recv1r_ref, recv1l_ref,
             recv2_ref,
             rs1r_send, rs1r_recv, rs1l_send, rs1l_recv,
             rs2_send, rs2_recv, ag2_send, ag2_recv,
             ag1r_send, ag1r_recv, ag1l_send, ag1l_recv):
        gg = lax.axis_index("i") // S
        q = scal_ref[0]
        right1 = scal_ref[1]
        left1 = scal_ref[2]
        right2 = scal_ref[3]
        left2 = scal_ref[4]

        barrier_sem = pltpu.get_barrier_semaphore()
        for nbr in (left1, right1, left2, right2):
            pl.semaphore_signal(
                barrier_sem, inc=1,
                device_id=(nbr,), device_id_type=pl.DeviceIdType.MESH,
            )
        pl.semaphore_wait(barrier_sem, 4)

        a = a_ref[...].astype(jnp.bfloat16)
        b = b_ref[...].astype(jnp.bfloat16)
        p_ref[...] = jnp.dot(
            a, b, preferred_element_type=jnp.float32
        ).astype(jnp.bfloat16)

        pending = []

        def quarter_copy(src, dst, send_sem, recv_sem, dev):
            d = pltpu.make_async_remote_copy(
                src_ref=src, dst_ref=dst, send_sem=send_sem,
                recv_sem=recv_sem, device_id=(dev,),
                device_id_type=pl.DeviceIdType.MESH,
            )
            d.start()
            pending.append(d)
            return d

        hops_r = []
        hops_l = []
        for h in range(S - 1):
            sr = lax.rem(q + S - h - 1, S)
            sl = lax.rem(q + h + 1, S)
            dr, dl = [], []
            for u in range(Q):
                top = pl.ds(sr * slab + u * qr, qr)
                bot = pl.ds(sl * slab + half + u * qr, qr)
                if h > 0:
                    hops_r[h - 1][u].wait_recv()
                    recv1r_ref[h - 1, u] = recv1r_ref[h - 1, u] + p_ref[top, :]
                    src_r = recv1r_ref.at[h - 1, u]
                else:
                    src_r = p_ref.at[top, :]
                dr.append(quarter_copy(
                    src_r, recv1r_ref.at[h, u],
                    rs1r_send.at[h, u], rs1r_recv.at[h, u], right1))
                if h > 0:
                    hops_l[h - 1][u].wait_recv()
                    recv1l_ref[h - 1, u] = recv1l_ref[h - 1, u] + p_ref[bot, :]
                    src_l = recv1l_ref.at[h - 1, u]
                else:
                    src_l = p_ref.at[bot, :]
                dl.append(quarter_copy(
                    src_l, recv1l_ref.at[h, u],
                    rs1l_send.at[h, u], rs1l_recv.at[h, u], left1))
            hops_r.append(dr)
            hops_l.append(dl)
        for u in range(Q):
            hops_r[S - 2][u].wait_recv()
            hops_l[S - 2][u].wait_recv()
            top = pl.ds(q * slab + u * qr, qr)
            bot = pl.ds(q * slab + half + u * qr, qr)
            p_ref[top, :] = p_ref[top, :] + recv1r_ref[S - 2, u]
            p_ref[bot, :] = p_ref[bot, :] + recv1l_ref[S - 2, u]

        base = q * slab
        rs2 = []
        for h in range(G - 1):
            s2 = lax.rem(gg + G - h - 1, G)
            own = pl.ds(base + s2 * sub, sub)
            if h > 0:
                rs2[h - 1].wait_recv()
                recv2_ref[h - 1] = recv2_ref[h - 1] + p_ref[own, :]
                src = recv2_ref.at[h - 1]
            else:
                src = p_ref.at[own, :]
            rs2.append(quarter_copy(
                src, recv2_ref.at[h],
                rs2_send.at[h], rs2_recv.at[h], right2))

        rs2[G - 2].wait_recv()
        z = (
            p_ref[pl.ds(base + gg * sub, sub), :].astype(jnp.float32)
            + recv2_ref[G - 2].astype(jnp.float32)
        )
        out_ref[pl.ds(base + gg * sub, sub), :] = _gelu(z).astype(jnp.bfloat16)

        ag2 = []
        for hh in range(G - 1):
            c2 = lax.rem(gg + G - hh, G)
            rows = pl.ds(base + c2 * sub, sub)
            if hh > 0:
                ag2[hh - 1].wait_recv()
            ag2.append(quarter_copy(
                out_ref.at[rows, :], out_ref.at[rows, :],
                ag2_send.at[hh], ag2_recv.at[hh], right2))
        ag2[G - 2].wait_recv()

        agr = []
        agl = []
        for hh in range(S - 1):
            cr = lax.rem(q + S - hh, S)
            cl = lax.rem(q + hh, S)
            dr, dl = [], []
            for u in range(Q):
                top = pl.ds(cr * slab + u * qr, qr)
                bot = pl.ds(cl * slab + half + u * qr, qr)
                if hh > 0:
                    agr[hh - 1][u].wait_recv()
                    agl[hh - 1][u].wait_recv()
                dr.append(quarter_copy(
                    out_ref.at[top, :], out_ref.at[top, :],
                    ag1r_send.at[hh, u], ag1r_recv.at[hh, u], right1))
                dl.append(quarter_copy(
                    out_ref.at[bot, :], out_ref.at[bot, :],
                    ag1l_send.at[hh, u], ag1l_recv.at[hh, u], left1))
            agr.append(dr)
            agl.append(dl)
        for u in range(Q):
            agr[S - 2][u].wait_recv()
            agl[S - 2][u].wait_recv()

        for d in pending:
            d.wait_send()

    return pl.pallas_call(
        body,
        out_shape=jax.ShapeDtypeStruct((m, n), jnp.bfloat16),
        in_specs=[
            pl.BlockSpec(memory_space=pltpu.SMEM),
            pl.BlockSpec(memory_space=pltpu.VMEM),
            pl.BlockSpec(memory_space=pltpu.VMEM),
        ],
        out_specs=pl.BlockSpec(memory_space=pltpu.VMEM),
        scratch_shapes=[
            pltpu.VMEM((m, n), jnp.bfloat16),
            pltpu.VMEM((S - 1, Q, qr, n), jnp.bfloat16),
            pltpu.VMEM((S - 1, Q, qr, n), jnp.bfloat16),
            pltpu.VMEM((G - 1, sub, n), jnp.bfloat16),
            pltpu.SemaphoreType.DMA((S - 1, Q)),
            pltpu.SemaphoreType.DMA((S - 1, Q)),
            pltpu.SemaphoreType.DMA((S - 1, Q)),
            pltpu.SemaphoreType.DMA((S - 1, Q)),
            pltpu.SemaphoreType.DMA((G - 1,)),
            pltpu.SemaphoreType.DMA((G - 1,)),
            pltpu.SemaphoreType.DMA((G - 1,)),
            pltpu.SemaphoreType.DMA((G - 1,)),
            pltpu.SemaphoreType.DMA((S - 1, Q)),
            pltpu.SemaphoreType.DMA((S - 1, Q)),
            pltpu.SemaphoreType.DMA((S - 1, Q)),
            pltpu.SemaphoreType.DMA((S - 1, Q)),
        ],
        compiler_params=pltpu.CompilerParams(collective_id=0),
    )(scal, A, B)


# device time: 82544 ns/iter; 3.9813x vs baseline; 1.0944x over previous
import jax
import jax.numpy as jnp
from jax import lax
from jax.experimental import pallas as pl
from jax.experimental.pallas import tpu as pltpu

W = 32
G = 4
S = 8
Q = 3
T = 3

PERM = (0, 1, 2, 5, 6, 7, 4, 3)
INV = (0, 1, 2, 7, 6, 3, 4, 5)


def _gelu(z):
    return 0.5 * z * (1.0 + jnp.tanh(0.7978845608 * (z + 0.044715 * z * z * z)))


def kernel(A, B):
    m, k = A.shape
    _, n = B.shape
    slab = m // S
    half = slab // 2
    qr = half // Q
    sub = slab // G
    th = sub // T

    r = lax.axis_index("i")
    g = r // S
    j = lax.rem(r, S)
    perm = jnp.array(PERM, dtype=jnp.int32)
    inv = jnp.array(INV, dtype=jnp.int32)
    q = inv[j]
    right1 = g * S + perm[lax.rem(q + 1, S)]
    left1 = g * S + perm[lax.rem(q + S - 1, S)]
    right2 = lax.rem(g + 1, G) * S + j
    left2 = lax.rem(g + G - 1, G) * S + j
    scal = jnp.stack([q, right1, left1, right2, left2]).astype(jnp.int32)

    def body(scal_ref, a_ref, b_ref, out_ref, p_ref, recv1r_ref, recv1l_ref,
             recv2_ref,
             rs1r_send, rs1r_recv, rs1l_send, rs1l_recv,
             rs2_send, rs2_recv, ag2_send, ag2_recv,
             ag1r_send, ag1r_recv, ag1l_send, ag1l_recv):
        gg = lax.axis_index("i") // S
        q = scal_ref[0]
        right1 = scal_ref[1]
        left1 = scal_ref[2]
        right2 = scal_ref[3]
        left2 = scal_ref[4]

        barrier_sem = pltpu.get_barrier_semaphore()
        for nbr in (left1, right1, left2, right2):
            pl.semaphore_signal(
                barrier_sem, inc=1,
                device_id=(nbr,), device_id_type=pl.DeviceIdType.MESH,
            )
        pl.semaphore_wait(barrier_sem, 4)

        bmat = b_ref[...].astype(jnp.bfloat16)

        def compute_slab(idx):
            rows = pl.ds(lax.rem(idx + 2 * S, S) * slab, slab)
            p_ref[rows, :] = jnp.dot(
                a_ref[rows, :].astype(jnp.bfloat16), bmat,
                preferred_element_type=jnp.float32,
            ).astype(jnp.bfloat16)

        pending = []

        def remote_copy(src, dst, send_sem, recv_sem, dev):
            d = pltpu.make_async_remote_copy(
                src_ref=src, dst_ref=dst, send_sem=send_sem,
                recv_sem=recv_sem, device_id=(dev,),
                device_id_type=pl.DeviceIdType.MESH,
            )
            d.start()
            pending.append(d)
            return d

        slab_schedule = {
            1: (q - 2, q + 2),
            2: (q - 3, q + 3),
            3: (q + 4,),
            4: (q,),
        }

        compute_slab(q - 1)
        compute_slab(q + 1)
        hops_r = []
        hops_l = []
        for h in range(S - 1):
            for idx in slab_schedule.get(h, ()):
                compute_slab(idx)
            sr = lax.rem(q + S - h - 1, S)
            sl = lax.rem(q + h + 1, S)
            dr, dl = [], []
            for u in range(Q):
                top = pl.ds(sr * slab + u * qr, qr)
                bot = pl.ds(sl * slab + half + u * qr, qr)
                if h > 0:
                    hops_r[h - 1][u].wait_recv()
                    recv1r_ref[h - 1, u] = recv1r_ref[h - 1, u] + p_ref[top, :]
                    src_r = recv1r_ref.at[h - 1, u]
                else:
                    src_r = p_ref.at[top, :]
                dr.append(remote_copy(
                    src_r, recv1r_ref.at[h, u],
                    rs1r_send.at[h, u], rs1r_recv.at[h, u], right1))
                if h > 0:
                    hops_l[h - 1][u].wait_recv()
                    recv1l_ref[h - 1, u] = recv1l_ref[h - 1, u] + p_ref[bot, :]
                    src_l = recv1l_ref.at[h - 1, u]
                else:
                    src_l = p_ref.at[bot, :]
                dl.append(remote_copy(
                    src_l, recv1l_ref.at[h, u],
                    rs1l_send.at[h, u], rs1l_recv.at[h, u], left1))
            hops_r.append(dr)
            hops_l.append(dl)
        for u in range(Q):
            hops_r[S - 2][u].wait_recv()
            hops_l[S - 2][u].wait_recv()
            top = pl.ds(q * slab + u * qr, qr)
            bot = pl.ds(q * slab + half + u * qr, qr)
            p_ref[top, :] = p_ref[top, :] + recv1r_ref[S - 2, u]
            p_ref[bot, :] = p_ref[bot, :] + recv1l_ref[S - 2, u]

        base = q * slab
        rs2 = []
        for h in range(G - 1):
            s2 = lax.rem(gg + G - h - 1, G)
            dt = []
            for t in range(T):
                rows = pl.ds(base + s2 * sub + t * th, th)
                if h > 0:
                    rs2[h - 1][t].wait_recv()
                    recv2_ref[h - 1, t] = recv2_ref[h - 1, t] + p_ref[rows, :]
                    src = recv2_ref.at[h - 1, t]
                else:
                    src = p_ref.at[rows, :]
                dt.append(remote_copy(
                    src, recv2_ref.at[h, t],
                    rs2_send.at[h, t], rs2_recv.at[h, t], right2))
            rs2.append(dt)

        ag2 = [[], [], []]
        for t in range(T):
            rs2[G - 2][t].wait_recv()
            rows = pl.ds(base + gg * sub + t * th, th)
            z = (
                p_ref[rows, :].astype(jnp.float32)
                + recv2_ref[G - 2, t].astype(jnp.float32)
            )
            out_ref[rows, :] = _gelu(z).astype(jnp.bfloat16)
            ag2[0].append(remote_copy(
                out_ref.at[rows, :], out_ref.at[rows, :],
                ag2_send.at[0, t], ag2_recv.at[0, t], right2))

        for hh in range(1, G - 1):
            c2 = lax.rem(gg + G - hh, G)
            for t in range(T):
                rows = pl.ds(base + c2 * sub + t * th, th)
                ag2[hh - 1][t].wait_recv()
                ag2[hh].append(remote_copy(
                    out_ref.at[rows, :], out_ref.at[rows, :],
                    ag2_send.at[hh, t], ag2_recv.at[hh, t], right2))
        for t in range(T):
            ag2[G - 2][t].wait_recv()

        agr = []
        agl = []
        for hh in range(S - 1):
            cr = lax.rem(q + S - hh, S)
            cl = lax.rem(q + hh, S)
            dr, dl = [], []
            for u in range(Q):
                top = pl.ds(cr * slab + u * qr, qr)
                bot = pl.ds(cl * slab + half + u * qr, qr)
                if hh > 0:
                    agr[hh - 1][u].wait_recv()
                    agl[hh - 1][u].wait_recv()
                dr.append(remote_copy(
                    out_ref.at[top, :], out_ref.at[top, :],
                    ag1r_send.at[hh, u], ag1r_recv.at[hh, u], right1))
                dl.append(remote_copy(
                    out_ref.at[bot, :], out_ref.at[bot, :],
                    ag1l_send.at[hh, u], ag1l_recv.at[hh, u], left1))
            agr.append(dr)
            agl.append(dl)
        for u in range(Q):
            agr[S - 2][u].wait_recv()
            agl[S - 2][u].wait_recv()

        for d in pending:
            d.wait_send()

    return pl.pallas_call(
        body,
        out_shape=jax.ShapeDtypeStruct((m, n), jnp.bfloat16),
        in_specs=[
            pl.BlockSpec(memory_space=pltpu.SMEM),
            pl.BlockSpec(memory_space=pltpu.VMEM),
            pl.BlockSpec(memory_space=pltpu.VMEM),
        ],
        out_specs=pl.BlockSpec(memory_space=pltpu.VMEM),
        scratch_shapes=[
            pltpu.VMEM((m, n), jnp.bfloat16),
            pltpu.VMEM((S - 1, Q, qr, n), jnp.bfloat16),
            pltpu.VMEM((S - 1, Q, qr, n), jnp.bfloat16),
            pltpu.VMEM((G - 1, T, th, n), jnp.bfloat16),
            pltpu.SemaphoreType.DMA((S - 1, Q)),
            pltpu.SemaphoreType.DMA((S - 1, Q)),
            pltpu.SemaphoreType.DMA((S - 1, Q)),
            pltpu.SemaphoreType.DMA((S - 1, Q)),
            pltpu.SemaphoreType.DMA((G - 1, T)),
            pltpu.SemaphoreType.DMA((G - 1, T)),
            pltpu.SemaphoreType.DMA((G - 1, T)),
            pltpu.SemaphoreType.DMA((G - 1, T)),
            pltpu.SemaphoreType.DMA((S - 1, Q)),
            pltpu.SemaphoreType.DMA((S - 1, Q)),
            pltpu.SemaphoreType.DMA((S - 1, Q)),
            pltpu.SemaphoreType.DMA((S - 1, Q)),
        ],
        compiler_params=pltpu.CompilerParams(collective_id=0),
    )(scal, A, B)


# device time: 79180 ns/iter; 4.1505x vs baseline; 1.0425x over previous
import jax
import jax.numpy as jnp
from jax import lax
from jax.experimental import pallas as pl
from jax.experimental.pallas import tpu as pltpu

W = 32
G = 4
S = 8
Q = 3
T = 3

PERM = (0, 1, 2, 5, 6, 7, 4, 3)
INV = (0, 1, 2, 7, 6, 3, 4, 5)


def _gelu(z):
    return 0.5 * z * (1.0 + jnp.tanh(0.7978845608 * (z + 0.044715 * z * z * z)))


def kernel(A, B):
    m, k = A.shape
    _, n = B.shape
    slab = m // S
    half = slab // 2
    qr = half // Q
    sub = slab // G
    th = sub // T

    perm_packed = 0
    inv_packed = 0
    for i in range(S):
        perm_packed |= PERM[i] << (4 * i)
        inv_packed |= INV[i] << (4 * i)

    def body(a_ref, b_ref, out_ref, p_ref, recv1r_ref, recv1l_ref,
             recv2_ref,
             rs1r_send, rs1r_recv, rs1l_send, rs1l_recv,
             rs2_send, rs2_recv, ag2_send, ag2_recv,
             ag1r_send, ag1r_recv, ag1l_send, ag1l_recv):
        r = lax.axis_index("i")
        gg = r // S
        jj = lax.rem(r, S)

        def nib(packed, idx):
            return lax.bitwise_and(
                lax.shift_right_logical(jnp.int32(packed), 4 * idx),
                jnp.int32(0xF),
            )

        q = nib(inv_packed, jj)
        right1 = gg * S + nib(perm_packed, lax.rem(q + 1, S))
        left1 = gg * S + nib(perm_packed, lax.rem(q + S - 1, S))
        right2 = lax.rem(gg + 1, G) * S + jj
        left2 = lax.rem(gg + G - 1, G) * S + jj

        barrier_sem = pltpu.get_barrier_semaphore()
        for nbr in (left1, right1, left2, right2):
            pl.semaphore_signal(
                barrier_sem, inc=1,
                device_id=(nbr,), device_id_type=pl.DeviceIdType.MESH,
            )
        pl.semaphore_wait(barrier_sem, 4)

        bmat = b_ref[...].astype(jnp.bfloat16)

        def compute_slab(idx):
            rows = pl.ds(lax.rem(idx + 2 * S, S) * slab, slab)
            p_ref[rows, :] = jnp.dot(
                a_ref[rows, :].astype(jnp.bfloat16), bmat,
                preferred_element_type=jnp.float32,
            ).astype(jnp.bfloat16)

        pending = []

        def remote_copy(src, dst, send_sem, recv_sem, dev):
            d = pltpu.make_async_remote_copy(
                src_ref=src, dst_ref=dst, send_sem=send_sem,
                recv_sem=recv_sem, device_id=(dev,),
                device_id_type=pl.DeviceIdType.MESH,
            )
            d.start()
            pending.append(d)
            return d

        slab_schedule = {
            1: (q - 2, q + 2),
            2: (q - 3, q + 3),
            3: (q + 4,),
            4: (q,),
        }

        compute_slab(q - 1)
        compute_slab(q + 1)
        hops_r = []
        hops_l = []
        for h in range(S - 1):
            for idx in slab_schedule.get(h, ()):
                compute_slab(idx)
            sr = lax.rem(q + S - h - 1, S)
            sl = lax.rem(q + h + 1, S)
            dr, dl = [], []
            for u in range(Q):
                top = pl.ds(sr * slab + u * qr, qr)
                bot = pl.ds(sl * slab + half + u * qr, qr)
                if h > 0:
                    hops_r[h - 1][u].wait_recv()
                    recv1r_ref[h - 1, u] = recv1r_ref[h - 1, u] + p_ref[top, :]
                    src_r = recv1r_ref.at[h - 1, u]
                else:
                    src_r = p_ref.at[top, :]
                dr.append(remote_copy(
                    src_r, recv1r_ref.at[h, u],
                    rs1r_send.at[h, u], rs1r_recv.at[h, u], right1))
                if h > 0:
                    hops_l[h - 1][u].wait_recv()
                    recv1l_ref[h - 1, u] = recv1l_ref[h - 1, u] + p_ref[bot, :]
                    src_l = recv1l_ref.at[h - 1, u]
                else:
                    src_l = p_ref.at[bot, :]
                dl.append(remote_copy(
                    src_l, recv1l_ref.at[h, u],
                    rs1l_send.at[h, u], rs1l_recv.at[h, u], left1))
            hops_r.append(dr)
            hops_l.append(dl)
        for u in range(Q):
            hops_r[S - 2][u].wait_recv()
            hops_l[S - 2][u].wait_recv()
            top = pl.ds(q * slab + u * qr, qr)
            bot = pl.ds(q * slab + half + u * qr, qr)
            p_ref[top, :] = p_ref[top, :] + recv1r_ref[S - 2, u]
            p_ref[bot, :] = p_ref[bot, :] + recv1l_ref[S - 2, u]

        base = q * slab
        rs2 = []
        for h in range(G - 1):
            s2 = lax.rem(gg + G - h - 1, G)
            dt = []
            for t in range(T):
                rows = pl.ds(base + s2 * sub + t * th, th)
                if h > 0:
                    rs2[h - 1][t].wait_recv()
                    recv2_ref[h - 1, t] = recv2_ref[h - 1, t] + p_ref[rows, :]
                    src = recv2_ref.at[h - 1, t]
                else:
                    src = p_ref.at[rows, :]
                dt.append(remote_copy(
                    src, recv2_ref.at[h, t],
                    rs2_send.at[h, t], rs2_recv.at[h, t], right2))
            rs2.append(dt)

        ag2 = [[], [], []]
        for t in range(T):
            rs2[G - 2][t].wait_recv()
            rows = pl.ds(base + gg * sub + t * th, th)
            z = (
                p_ref[rows, :].astype(jnp.float32)
                + recv2_ref[G - 2, t].astype(jnp.float32)
            )
            out_ref[rows, :] = _gelu(z).astype(jnp.bfloat16)
            ag2[0].append(remote_copy(
                out_ref.at[rows, :], out_ref.at[rows, :],
                ag2_send.at[0, t], ag2_recv.at[0, t], right2))

        for hh in range(1, G - 1):
            c2 = lax.rem(gg + G - hh, G)
            for t in range(T):
                rows = pl.ds(base + c2 * sub + t * th, th)
                ag2[hh - 1][t].wait_recv()
                ag2[hh].append(remote_copy(
                    out_ref.at[rows, :], out_ref.at[rows, :],
                    ag2_send.at[hh, t], ag2_recv.at[hh, t], right2))
        for t in range(T):
            ag2[G - 2][t].wait_recv()

        agr = []
        agl = []
        for hh in range(S - 1):
            cr = lax.rem(q + S - hh, S)
            cl = lax.rem(q + hh, S)
            dr, dl = [], []
            for u in range(Q):
                top = pl.ds(cr * slab + u * qr, qr)
                bot = pl.ds(cl * slab + half + u * qr, qr)
                if hh > 0:
                    agr[hh - 1][u].wait_recv()
                    agl[hh - 1][u].wait_recv()
                dr.append(remote_copy(
                    out_ref.at[top, :], out_ref.at[top, :],
                    ag1r_send.at[hh, u], ag1r_recv.at[hh, u], right1))
                dl.append(remote_copy(
                    out_ref.at[bot, :], out_ref.at[bot, :],
                    ag1l_send.at[hh, u], ag1l_recv.at[hh, u], left1))
            agr.append(dr)
            agl.append(dl)
        for u in range(Q):
            agr[S - 2][u].wait_recv()
            agl[S - 2][u].wait_recv()

        for d in pending:
            d.wait_send()

    return pl.pallas_call(
        body,
        out_shape=jax.ShapeDtypeStruct((m, n), jnp.bfloat16),
        in_specs=[
            pl.BlockSpec(memory_space=pltpu.VMEM),
            pl.BlockSpec(memory_space=pltpu.VMEM),
        ],
        out_specs=pl.BlockSpec(memory_space=pltpu.VMEM),
        scratch_shapes=[
            pltpu.VMEM((m, n), jnp.bfloat16),
            pltpu.VMEM((S - 1, Q, qr, n), jnp.bfloat16),
            pltpu.VMEM((S - 1, Q, qr, n), jnp.bfloat16),
            pltpu.VMEM((G - 1, T, th, n), jnp.bfloat16),
            pltpu.SemaphoreType.DMA((S - 1, Q)),
            pltpu.SemaphoreType.DMA((S - 1, Q)),
            pltpu.SemaphoreType.DMA((S - 1, Q)),
            pltpu.SemaphoreType.DMA((S - 1, Q)),
            pltpu.SemaphoreType.DMA((G - 1, T)),
            pltpu.SemaphoreType.DMA((G - 1, T)),
            pltpu.SemaphoreType.DMA((G - 1, T)),
            pltpu.SemaphoreType.DMA((G - 1, T)),
            pltpu.SemaphoreType.DMA((S - 1, Q)),
            pltpu.SemaphoreType.DMA((S - 1, Q)),
            pltpu.SemaphoreType.DMA((S - 1, Q)),
            pltpu.SemaphoreType.DMA((S - 1, Q)),
        ],
        compiler_params=pltpu.CompilerParams(collective_id=0),
    )(A, B)
